# Initial kernel scaffold; baseline (speedup 1.0000x reference)
import jax
import jax.numpy as jnp
from jax.experimental import pallas as pl


def kernel(x, edge_index, W_self1, W_neigh1, b1, W_self2, W_neigh2, b2, W_fc, b_fc):
    raise NotImplementedError("write your pallas kernel here")


# speedup vs baseline: 7.3623x; 7.3623x over previous
import functools

import jax
import jax.numpy as jnp
from jax import lax
from jax.experimental import pallas as pl
from jax.experimental.pallas import tpu as pltpu
from jax.experimental.pallas import tpu_sc as plsc

N_NODES = 10000
N_EDGES = 320000
D = 128
N_CLS = 40

NC = 2
NS = 16
NW = NC * NS
EPW = N_EDGES // NW
CHUNK = 80
NCHUNK = EPW // CHUNK
RPT = 624
TAIL = N_NODES - NS * RPT

_mesh = plsc.VectorSubcoreMesh(core_axis_name="c", subcore_axis_name="s")


def _sc_agg_body(with_deg, *refs):
    if with_deg:
        (x_hbm, src_hbm, dst_hbm, z2_hbm, z1_hbm,
         agg_hbm, deg_hbm,
         src_v, dst_v, rows_v, ones_v, acc_sh, deg_sh, sem) = refs
    else:
        (x_hbm, src_hbm, dst_hbm, z2_hbm,
         agg_hbm,
         src_v, dst_v, rows_v, acc_sh, sem) = refs

    cid = lax.axis_index("c")
    sid = lax.axis_index("s")
    wid = cid * NS + sid

    pltpu.sync_copy(z2_hbm.at[pl.ds(sid * RPT, RPT)],
                    acc_sh.at[pl.ds(sid * RPT, RPT)])

    @pl.when(sid == NS - 1)
    def _():
        pltpu.sync_copy(z2_hbm.at[pl.ds(NS * RPT, TAIL)],
                        acc_sh.at[pl.ds(NS * RPT, TAIL)])
    if with_deg:
        @pl.when(sid == 0)
        def _():
            pltpu.sync_copy(z1_hbm, deg_sh)
        for i in range(CHUNK // 16):
            ones_v[pl.ds(i * 16, 16)] = jnp.full((16,), 1.0, jnp.float32)

    pltpu.sync_copy(src_hbm.at[wid], src_v)
    pltpu.sync_copy(dst_hbm.at[wid], dst_v)

    plsc.subcore_barrier()

    def step(j, carry):
        pltpu.async_copy(x_hbm.at[src_v.at[j]], rows_v, sem).wait()
        pltpu.sync_copy(rows_v, acc_sh.at[dst_v.at[j]], add=True)
        if with_deg:
            pltpu.sync_copy(ones_v, deg_sh.at[dst_v.at[j]], add=True)
        return carry

    lax.fori_loop(0, NCHUNK, step, 0)

    plsc.subcore_barrier()

    pltpu.sync_copy(acc_sh.at[pl.ds(sid * RPT, RPT)],
                    agg_hbm.at[cid, pl.ds(sid * RPT, RPT)])

    @pl.when(sid == NS - 1)
    def _():
        pltpu.sync_copy(acc_sh.at[pl.ds(NS * RPT, TAIL)],
                        agg_hbm.at[cid, pl.ds(NS * RPT, TAIL)])
    if with_deg:
        @pl.when(sid == 0)
        def _():
            pltpu.sync_copy(deg_sh, deg_hbm.at[cid])


_sc_agg_deg = functools.partial(
    pl.kernel,
    functools.partial(_sc_agg_body, True),
    mesh=_mesh,
    out_type=[
        jax.ShapeDtypeStruct((NC, N_NODES, D), jnp.float32),
        jax.ShapeDtypeStruct((NC, N_NODES), jnp.float32),
    ],
    scratch_types=[
        pltpu.VMEM((NCHUNK, CHUNK), jnp.int32),
        pltpu.VMEM((NCHUNK, CHUNK), jnp.int32),
        pltpu.VMEM((CHUNK, D), jnp.float32),
        pltpu.VMEM((CHUNK,), jnp.float32),
        pltpu.VMEM_SHARED((N_NODES, D), jnp.float32),
        pltpu.VMEM_SHARED((N_NODES,), jnp.float32),
        pltpu.SemaphoreType.DMA,
    ],
)()

_sc_agg = functools.partial(
    pl.kernel,
    functools.partial(_sc_agg_body, False),
    mesh=_mesh,
    out_type=jax.ShapeDtypeStruct((NC, N_NODES, D), jnp.float32),
    scratch_types=[
        pltpu.VMEM((NCHUNK, CHUNK), jnp.int32),
        pltpu.VMEM((NCHUNK, CHUNK), jnp.int32),
        pltpu.VMEM((CHUNK, D), jnp.float32),
        pltpu.VMEM_SHARED((N_NODES, D), jnp.float32),
        pltpu.SemaphoreType.DMA,
    ],
)()


BLK = 1000


def _tc_layer_body(final, *refs):
    if final:
        (h_ref, a0_ref, a1_ref, d0_ref, d1_ref,
         ws_ref, wn_ref, b_ref, wfc_ref, bfc_ref, o_ref) = refs
    else:
        (h_ref, a0_ref, a1_ref, d0_ref, d1_ref,
         ws_ref, wn_ref, b_ref, o_ref) = refs
    deg = d0_ref[...] + d1_ref[...]
    inv = 1.0 / jnp.maximum(deg, 1.0)
    hn = (a0_ref[...] + a1_ref[...]) * inv
    h = (jnp.dot(h_ref[...], ws_ref[...], preferred_element_type=jnp.float32)
         + jnp.dot(hn, wn_ref[...], preferred_element_type=jnp.float32)
         + b_ref[...])
    h = jnp.maximum(h, 0.0)
    if final:
        h = (jnp.dot(h, wfc_ref[...], preferred_element_type=jnp.float32)
             + bfc_ref[...])
    o_ref[...] = h


def _row_spec():
    return pl.BlockSpec((BLK, D), lambda i: (i, 0))


def _tc_layer(final):
    n_in = 10 if final else 8
    in_specs = [
        _row_spec(),
        _row_spec(),
        _row_spec(),
        pl.BlockSpec((BLK, 1), lambda i: (i, 0)),
        pl.BlockSpec((BLK, 1), lambda i: (i, 0)),
        pl.BlockSpec((D, D), lambda i: (0, 0)),
        pl.BlockSpec((D, D), lambda i: (0, 0)),
        pl.BlockSpec((1, D), lambda i: (0, 0)),
    ]
    if final:
        in_specs += [
            pl.BlockSpec((D, D), lambda i: (0, 0)),
            pl.BlockSpec((1, D), lambda i: (0, 0)),
        ]
    assert len(in_specs) == n_in
    return pl.pallas_call(
        functools.partial(_tc_layer_body, final),
        grid=(N_NODES // BLK,),
        in_specs=in_specs,
        out_specs=_row_spec(),
        out_shape=jax.ShapeDtypeStruct((N_NODES, D), jnp.float32),
    )


_tc_mid = _tc_layer(False)
_tc_fin = _tc_layer(True)


def kernel(x, edge_index, W_self1, W_neigh1, b1, W_self2, W_neigh2, b2,
           W_fc, b_fc):
    src = edge_index[0].astype(jnp.int32).reshape(NW, NCHUNK, CHUNK)
    dst = edge_index[1].astype(jnp.int32).reshape(NW, NCHUNK, CHUNK)
    z2 = jnp.zeros((N_NODES, D), jnp.float32)
    z1 = jnp.zeros((N_NODES,), jnp.float32)

    agg1, degp = _sc_agg_deg(x, src, dst, z2, z1)
    d0 = degp[0].reshape(N_NODES, 1)
    d1 = degp[1].reshape(N_NODES, 1)
    b1r = b1.reshape(1, D)
    b2r = b2.reshape(1, D)

    h1 = _tc_mid(x, agg1[0], agg1[1], d0, d1, W_self1, W_neigh1, b1r)

    agg2 = _sc_agg(h1, src, dst, z2)

    wfc_pad = jnp.zeros((D, D), jnp.float32).at[:, :N_CLS].set(W_fc)
    bfc_pad = jnp.zeros((1, D), jnp.float32).at[0, :N_CLS].set(b_fc)
    out_pad = _tc_fin(h1, agg2[0], agg2[1], d0, d1, W_self2, W_neigh2, b2r,
                      wfc_pad, bfc_pad)
    return out_pad[:, :N_CLS]
